# Initial kernel scaffold; baseline (speedup 1.0000x reference)
import jax
import jax.numpy as jnp
from jax.experimental import pallas as pl


def kernel(x, edge_index, W1, a_src1, a_dst1, b1, W2, a_src2, a_dst2, b2):
    raise NotImplementedError("write your pallas kernel here")


# speedup vs baseline: 21.9027x; 21.9027x over previous
import functools

import jax
import jax.numpy as jnp
from jax import lax
from jax.experimental import pallas as pl
from jax.experimental.pallas import tpu as pltpu
from jax.experimental.pallas import tpu_sc as plsc

N = 10000
E = 320000
D = 128
NW = 32
EPT = E // NW
K = 128
NB = -(-EPT // K)
EPT_PAD = NB * K
RPT = 624


def _tc_in_body(x_ref, w_ref, aa_ref, h_ref, sl_ref):
    h = jnp.dot(x_ref[...], w_ref[...], preferred_element_type=jnp.float32)
    h_ref[...] = h
    sl_ref[...] = jnp.dot(h, aa_ref[...], preferred_element_type=jnp.float32)


def _tc_in(x, w, aa):
    return pl.pallas_call(
        _tc_in_body,
        out_shape=[
            jax.ShapeDtypeStruct((N, D), jnp.float32),
            jax.ShapeDtypeStruct((N, 2), jnp.float32),
        ],
    )(x, w, aa)


def _combine(acc_ref, den_ref, mx_ref, b_ref):
    mx = mx_ref[...].reshape(16, 2, 16)
    m_sc = jnp.max(mx, axis=(0, 2))
    gamma = jnp.exp(m_sc - jnp.max(m_sc))
    den = jnp.sum(den_ref[...].reshape(16, 2, N) * gamma[None, :, None],
                  axis=(0, 1))
    acc = acc_ref[0] * gamma[0] + acc_ref[1] * gamma[1]
    z = acc / (den[:, None] + 1e-16) + b_ref[...][None, :]
    return jnp.maximum(z, 0.0)


def _tc_mid_body(acc_ref, den_ref, mx_ref, b_ref, w_ref, aa_ref, h_ref, sl_ref):
    z = _combine(acc_ref, den_ref, mx_ref, b_ref)
    h = jnp.dot(z, w_ref[...], preferred_element_type=jnp.float32)
    h_ref[...] = h
    sl_ref[...] = jnp.dot(h, aa_ref[...], preferred_element_type=jnp.float32)


def _tc_mid(acc, den, mx, b, w, aa):
    return pl.pallas_call(
        _tc_mid_body,
        out_shape=[
            jax.ShapeDtypeStruct((N, D), jnp.float32),
            jax.ShapeDtypeStruct((N, 2), jnp.float32),
        ],
    )(acc, den, mx, b, w, aa)


def _tc_out_body(acc_ref, den_ref, mx_ref, b_ref, out_ref):
    z = _combine(acc_ref, den_ref, mx_ref, b_ref)
    out_ref[...] = jnp.mean(z, axis=0, keepdims=True)


def _tc_out(acc, den, mx, b):
    return pl.pallas_call(
        _tc_out_body,
        out_shape=jax.ShapeDtypeStruct((1, D), jnp.float32),
    )(acc, den, mx, b)


def _sc_logits_body(src_hbm, dst_hbm, as_hbm, ad_hbm,
                    ex_out, den_out, max_out,
                    src_v, dst_v, ex_v, as_v, ad_v, den_v, m16_v,
                    m16r_v, mxall_v, mx_sh):
    c = lax.axis_index("c")
    s = lax.axis_index("s")
    wid = s * 2 + c

    def zden(i, _):
        den_v[pl.ds(i * 16, 16)] = jnp.zeros((16,), jnp.float32)
        return 0
    lax.fori_loop(0, N // 16, zden, 0)

    pltpu.sync_copy(src_hbm.at[wid], src_v)
    pltpu.sync_copy(dst_hbm.at[wid], dst_v)
    pltpu.sync_copy(as_hbm, as_v)
    pltpu.sync_copy(ad_hbm, ad_v)

    def a1(j, m):
        for cc in range(K // 16):
            sl = pl.ds(cc * 16, 16)
            si = src_v[j, sl]
            di = dst_v[j, sl]
            e = plsc.load_gather(as_v, [si]) + plsc.load_gather(ad_v, [di])
            e = jnp.where(e >= 0, e, 0.2 * e)
            ex_v[j, sl] = e
            m = jnp.maximum(m, e)
        return m
    m16 = lax.fori_loop(0, NB, a1, jnp.full((16,), -3e38, jnp.float32))

    m16_v[pl.ds(0, 16)] = m16
    m16r_v[0, pl.ds(0, 16)] = m16
    pltpu.sync_copy(m16_v, max_out.at[pl.ds(wid * 16, 16)])
    pltpu.sync_copy(m16r_v, mx_sh.at[s])
    plsc.subcore_barrier()
    pltpu.sync_copy(mx_sh, mxall_v)
    m = mxall_v[0, 0, pl.ds(0, 16)]
    for i in range(1, 16):
        m = jnp.maximum(m, mxall_v[i, 0, pl.ds(0, 16)])
    m_sc = jnp.max(m)

    def a3(j, _):
        for cc in range(K // 16):
            sl = pl.ds(cc * 16, 16)
            base = j * K + cc * 16
            valid = jnp.full((16,), base, jnp.int32) < EPT
            ex = jnp.where(valid, jnp.exp(ex_v[j, sl] - m_sc), 0.0)
            ex_v[j, sl] = ex
            plsc.addupdate_scatter(den_v, [dst_v[j, sl]], ex)
        return 0
    lax.fori_loop(0, NB, a3, 0)
    pltpu.sync_copy(den_v, den_out.at[pl.ds(wid * N, N)])
    pltpu.sync_copy(ex_v, ex_out.at[wid])


_sc_logits = functools.partial(
    pl.kernel,
    out_type=[
        jax.ShapeDtypeStruct((NW, NB, K), jnp.float32),
        jax.ShapeDtypeStruct((NW * N,), jnp.float32),
        jax.ShapeDtypeStruct((NW * 16,), jnp.float32),
    ],
    mesh=plsc.VectorSubcoreMesh(core_axis_name="c", subcore_axis_name="s"),
    compiler_params=pltpu.CompilerParams(needs_layout_passes=False),
    scratch_types=[
        pltpu.VMEM((NB, K), jnp.int32),
        pltpu.VMEM((NB, K), jnp.int32),
        pltpu.VMEM((NB, K), jnp.float32),
        pltpu.VMEM((N,), jnp.float32),
        pltpu.VMEM((N,), jnp.float32),
        pltpu.VMEM((N,), jnp.float32),
        pltpu.VMEM((16,), jnp.float32),
        pltpu.VMEM((1, 16), jnp.float32),
        pltpu.VMEM((16, 1, 16), jnp.float32),
        pltpu.VMEM_SHARED((16, 1, 16), jnp.float32),
    ],
)(_sc_logits_body)


def _sc_aggregate_body(src_hbm, dst_hbm, ex_hbm, h_hbm,
                       acc_out,
                       src_v, dst_v, ex_v, row_v, acc_sh, sem):
    c = lax.axis_index("c")
    s = lax.axis_index("s")
    wid = s * 2 + c

    def zrow(r, _):
        for cc in range(D // 16):
            row_v[r, pl.ds(cc * 16, 16)] = jnp.zeros((16,), jnp.float32)
        return 0
    lax.fori_loop(0, K, zrow, 0)
    for off, sz in ((0, 128), (128, 128), (256, 128), (384, 128), (496, 128)):
        pltpu.sync_copy(row_v.at[pl.ds(0, sz)],
                        acc_sh.at[pl.ds(s * RPT + off, sz)])

    @pl.when(s == 15)
    def _():
        pltpu.sync_copy(row_v.at[pl.ds(0, 16)],
                        acc_sh.at[pl.ds(N - 16, 16)])

    pltpu.sync_copy(src_hbm.at[wid], src_v)
    pltpu.sync_copy(dst_hbm.at[wid], dst_v)
    pltpu.sync_copy(ex_hbm.at[wid], ex_v)
    plsc.subcore_barrier()

    def b(j, _):
        pltpu.async_copy(h_hbm.at[src_v.at[j]], row_v, sem).wait()

        def scale(r, _):
            v = plsc.load_gather(ex_v, [jnp.full((16,), j, jnp.int32),
                                        jnp.full((16,), r, jnp.int32)])
            for cc in range(D // 16):
                sl = pl.ds(cc * 16, 16)
                row_v[r, sl] = row_v[r, sl] * v
            return 0
        lax.fori_loop(0, K, scale, 0)
        pltpu.sync_copy(row_v, acc_sh.at[dst_v.at[j]], add=True)
        return 0
    lax.fori_loop(0, NB, b, 0)

    plsc.subcore_barrier()
    pltpu.sync_copy(acc_sh.at[pl.ds(s * RPT, RPT)],
                    acc_out.at[c, pl.ds(s * RPT, RPT)])

    @pl.when(s == 15)
    def _():
        pltpu.sync_copy(acc_sh.at[pl.ds(N - 16, 16)],
                        acc_out.at[c, pl.ds(N - 16, 16)])


_sc_aggregate = functools.partial(
    pl.kernel,
    out_type=jax.ShapeDtypeStruct((2, N, D), jnp.float32),
    mesh=plsc.VectorSubcoreMesh(core_axis_name="c", subcore_axis_name="s"),
    compiler_params=pltpu.CompilerParams(needs_layout_passes=False),
    scratch_types=[
        pltpu.VMEM((NB, K), jnp.int32),
        pltpu.VMEM((NB, K), jnp.int32),
        pltpu.VMEM((NB, K), jnp.float32),
        pltpu.VMEM((K, D), jnp.float32),
        pltpu.VMEM_SHARED((N, D), jnp.float32),
        pltpu.SemaphoreType.DMA,
    ],
)(_sc_aggregate_body)


def _sc_edge(srcp, dstp, h, as_, ad_):
    ex, den, mx = _sc_logits(srcp, dstp, as_, ad_)
    acc = _sc_aggregate(srcp, dstp, ex, h)
    return acc, den, mx


def kernel(x, edge_index, W1, a_src1, a_dst1, b1, W2, a_src2, a_dst2, b2):
    src = edge_index[0].astype(jnp.int32)
    dst = edge_index[1].astype(jnp.int32)
    pad = jnp.zeros((NW, EPT_PAD - EPT), jnp.int32)
    srcp = jnp.concatenate([src.reshape(NW, EPT), pad], axis=1).reshape(NW, NB, K)
    dstp = jnp.concatenate([dst.reshape(NW, EPT), pad], axis=1).reshape(NW, NB, K)

    aa1 = jnp.stack([a_src1, a_dst1], axis=1)
    aa2 = jnp.stack([a_src2, a_dst2], axis=1)

    h1, sl1 = _tc_in(x, W1, aa1)
    acc1, den1, mx1 = _sc_edge(srcp, dstp, h1, sl1[:, 0] + 0.0, sl1[:, 1] + 0.0)
    h2, sl2 = _tc_mid(acc1, den1.reshape(NW, N), mx1.reshape(NW, 16), b1, W2, aa2)
    acc2, den2, mx2 = _sc_edge(srcp, dstp, h2, sl2[:, 0] + 0.0, sl2[:, 1] + 0.0)
    return _tc_out(acc2, den2.reshape(NW, N), mx2.reshape(NW, 16), b2)
